# baseline (speedup 1.0000x reference)
import jax
import jax.numpy as jnp
from jax.experimental import pallas as pl
from jax.experimental.pallas import tpu as pltpu

_S = 200
_THRESH = 0.85
_TB = 128


def _cell_kernel(x_ref, slots_ref, ptr_ref, wv_ref, bv_ref, we_ref, be_ref,
                 pos_ref, sw_ref, wp_ref, bp_ref,
                 h_ref, out_slots_ref, out_ptr_ref):
    x = x_ref[...]
    logit = jnp.sum(x * we_ref[...], axis=1, keepdims=True) + be_ref[...]
    mask = jax.nn.sigmoid(logit) > _THRESH
    v = jax.lax.dot_general(x, wv_ref[...], (((1,), (1,)), ((), ())),
                            preferred_element_type=jnp.float32) + bv_ref[...]
    slots = slots_ref[...]
    ptr = ptr_ref[...]
    sel = jax.lax.broadcasted_iota(jnp.int32, (_TB, _S), 1) == ptr
    sel = jnp.logical_and(sel, mask)
    selF = sel.astype(jnp.float32)[:, :, None]
    v3 = v[:, None, :]
    new_slots = jnp.where(selF > 0.5, v3, slots)
    out_slots_ref[...] = new_slots
    sw = sw_ref[...]
    ew = jnp.exp(sw - jnp.max(sw, axis=1, keepdims=True))
    w = ew / jnp.sum(ew, axis=1, keepdims=True)
    pos_c = jax.lax.dot_general(w, pos_ref[...], (((1,), (0,)), ((), ())),
                                preferred_element_type=jnp.float32)
    w3 = jax.lax.broadcast_in_dim(w, (_TB, 1, _S), (1, 2))
    fused3 = jax.lax.dot_general(w3, new_slots, (((2,), (1,)), ((0,), (0,))),
                                 preferred_element_type=jnp.float32)
    fused = jnp.sum(fused3, axis=1) + pos_c
    h = jax.lax.dot_general(fused, wp_ref[...], (((1,), (1,)), ((), ())),
                            preferred_element_type=jnp.float32) + bp_ref[...]
    h_ref[...] = h
    out_ptr_ref[...] = jnp.where(
        jnp.logical_and(mask, ptr == _S - 1), 0, ptr + mask.astype(ptr.dtype))


def kernel(x_t, slots, ptr, W_value, b_value, W_event, b_event, pos_emb,
           slot_weights, W_proj, b_proj, *, interpret=False):
    B, D = x_t.shape
    S = slots.shape[1]
    H = W_proj.shape[0]
    TB = _TB
    ptr2 = ptr.reshape(B, 1)
    h, new_slots, new_ptr2 = pl.pallas_call(
        _cell_kernel,
        grid=(B // TB,),
        in_specs=[
            pl.BlockSpec((TB, D), lambda i: (i, 0)),
            pl.BlockSpec((TB, S, D), lambda i: (i, 0, 0)),
            pl.BlockSpec((TB, 1), lambda i: (i, 0)),
            pl.BlockSpec((D, D), lambda i: (0, 0)),
            pl.BlockSpec((1, D), lambda i: (0, 0)),
            pl.BlockSpec((1, D), lambda i: (0, 0)),
            pl.BlockSpec((1, 1), lambda i: (0, 0)),
            pl.BlockSpec((S, D), lambda i: (0, 0)),
            pl.BlockSpec((1, S), lambda i: (0, 0)),
            pl.BlockSpec((H, D), lambda i: (0, 0)),
            pl.BlockSpec((1, H), lambda i: (0, 0)),
        ],
        out_specs=[
            pl.BlockSpec((TB, H), lambda i: (i, 0)),
            pl.BlockSpec((TB, S, D), lambda i: (i, 0, 0)),
            pl.BlockSpec((TB, 1), lambda i: (i, 0)),
        ],
        out_shape=[
            jax.ShapeDtypeStruct((B, H), jnp.float32),
            jax.ShapeDtypeStruct((B, S, D), jnp.float32),
            jax.ShapeDtypeStruct((B, 1), ptr.dtype),
        ],
        compiler_params=pltpu.CompilerParams(
            dimension_semantics=("arbitrary",)),
        interpret=interpret,
    )(x_t, slots, ptr2, W_value, b_value.reshape(1, D), W_event,
      b_event.reshape(1, 1), pos_emb, slot_weights.reshape(1, S), W_proj,
      b_proj.reshape(1, H))
    return h, new_slots, new_ptr2.reshape(B)


# speedup vs baseline: 1.0079x; 1.0079x over previous
import jax
import jax.numpy as jnp
from jax.experimental import pallas as pl
from jax.experimental.pallas import tpu as pltpu

_S = 200
_THRESH = 0.85
_TB = 64


def _cell_kernel(x_ref, slots_ref, ptr_ref, wv_ref, bv_ref, we_ref, be_ref,
                 pos_ref, sw_ref, wp_ref, bp_ref,
                 h_ref, out_slots_ref, out_ptr_ref):
    i = pl.program_id(0)
    r0 = i * _TB
    x = x_ref[pl.ds(r0, _TB), :]
    logit = jnp.sum(x * we_ref[...], axis=1, keepdims=True) + be_ref[...]
    mask = jax.nn.sigmoid(logit) > _THRESH
    v = jax.lax.dot_general(x, wv_ref[...], (((1,), (1,)), ((), ())),
                            preferred_element_type=jnp.float32) + bv_ref[...]
    slots = slots_ref[...]
    ptr = ptr_ref[pl.ds(r0, _TB), :]
    sel = jax.lax.broadcasted_iota(jnp.int32, (_TB, _S), 1) == ptr
    sel = jnp.logical_and(sel, mask)
    selF = sel.astype(jnp.float32)[:, :, None]
    v3 = v[:, None, :]
    new_slots = jnp.where(selF > 0.5, v3, slots)
    out_slots_ref[...] = new_slots
    sw = sw_ref[...]
    ew = jnp.exp(sw - jnp.max(sw, axis=1, keepdims=True))
    w = ew / jnp.sum(ew, axis=1, keepdims=True)
    pos_c = jax.lax.dot_general(w, pos_ref[...], (((1,), (0,)), ((), ())),
                                preferred_element_type=jnp.float32)
    w3 = jax.lax.broadcast_in_dim(w, (_TB, 1, _S), (1, 2))
    fused3 = jax.lax.dot_general(w3, new_slots, (((2,), (1,)), ((0,), (0,))),
                                 preferred_element_type=jnp.float32)
    fused = jnp.sum(fused3, axis=1) + pos_c
    h = jax.lax.dot_general(fused, wp_ref[...], (((1,), (1,)), ((), ())),
                            preferred_element_type=jnp.float32) + bp_ref[...]
    h_ref[pl.ds(r0, _TB), :] = h
    out_ptr_ref[pl.ds(r0, _TB), :] = jnp.where(
        jnp.logical_and(mask, ptr == _S - 1), 0, ptr + mask.astype(ptr.dtype))


def kernel(x_t, slots, ptr, W_value, b_value, W_event, b_event, pos_emb,
           slot_weights, W_proj, b_proj, *, interpret=False):
    B, D = x_t.shape
    S = slots.shape[1]
    H = W_proj.shape[0]
    TB = _TB
    ptr2 = ptr.reshape(B, 1)
    h, new_slots, new_ptr2 = pl.pallas_call(
        _cell_kernel,
        grid=(B // TB,),
        in_specs=[
            pl.BlockSpec((B, D), lambda i: (0, 0)),
            pl.BlockSpec((TB, S, D), lambda i: (i, 0, 0)),
            pl.BlockSpec((B, 1), lambda i: (0, 0)),
            pl.BlockSpec((D, D), lambda i: (0, 0)),
            pl.BlockSpec((1, D), lambda i: (0, 0)),
            pl.BlockSpec((1, D), lambda i: (0, 0)),
            pl.BlockSpec((1, 1), lambda i: (0, 0)),
            pl.BlockSpec((S, D), lambda i: (0, 0)),
            pl.BlockSpec((1, S), lambda i: (0, 0)),
            pl.BlockSpec((H, D), lambda i: (0, 0)),
            pl.BlockSpec((1, H), lambda i: (0, 0)),
        ],
        out_specs=[
            pl.BlockSpec((B, H), lambda i: (0, 0)),
            pl.BlockSpec((TB, S, D), lambda i: (i, 0, 0)),
            pl.BlockSpec((B, 1), lambda i: (0, 0)),
        ],
        out_shape=[
            jax.ShapeDtypeStruct((B, H), jnp.float32),
            jax.ShapeDtypeStruct((B, S, D), jnp.float32),
            jax.ShapeDtypeStruct((B, 1), ptr.dtype),
        ],
        compiler_params=pltpu.CompilerParams(
            dimension_semantics=("parallel",)),
        interpret=interpret,
    )(x_t, slots, ptr2, W_value, b_value.reshape(1, D), W_event,
      b_event.reshape(1, 1), pos_emb, slot_weights.reshape(1, S), W_proj,
      b_proj.reshape(1, H))
    return h, new_slots, new_ptr2.reshape(B)


# speedup vs baseline: 1.0082x; 1.0003x over previous
import jax
import jax.numpy as jnp
from jax.experimental import pallas as pl
from jax.experimental.pallas import tpu as pltpu

_S = 200
_THRESH = 0.85
_TB = 64


def _cell_kernel(x_ref, slots_ref, ptr_ref, wv_ref, bv_ref, we_ref, be_ref,
                 pos_ref, sw_ref, wp_ref, bp_ref,
                 h_ref, out_slots_ref, out_ptr_ref):
    i = pl.program_id(0)
    r0 = i * _TB
    x = x_ref[pl.ds(r0, _TB), :]
    logit = jnp.sum(x * we_ref[...], axis=1, keepdims=True) + be_ref[...]
    mask = jax.nn.sigmoid(logit) > _THRESH
    v = jax.lax.dot_general(x, wv_ref[...], (((1,), (1,)), ((), ())),
                            preferred_element_type=jnp.float32) + bv_ref[...]
    slots = slots_ref[...]
    ptr = ptr_ref[pl.ds(r0, _TB), :]
    sel = jax.lax.broadcasted_iota(jnp.int32, (_TB, _S), 1) == ptr
    sel = jnp.logical_and(sel, mask)
    selF = sel.astype(jnp.float32)[:, :, None]
    v3 = v[:, None, :]
    new_slots = jnp.where(selF > 0.5, v3, slots)
    out_slots_ref[...] = new_slots
    sw = sw_ref[...]
    ew = jnp.exp(sw - jnp.max(sw, axis=1, keepdims=True))
    w = ew / jnp.sum(ew, axis=1, keepdims=True)
    pos_c = jax.lax.dot_general(w, pos_ref[...], (((1,), (0,)), ((), ())),
                                preferred_element_type=jnp.float32)
    w3 = jax.lax.broadcast_in_dim(w, (_TB, 1, _S), (1, 2))
    fused3 = jax.lax.dot_general(w3, new_slots, (((2,), (1,)), ((0,), (0,))),
                                 preferred_element_type=jnp.float32)
    fused = jnp.sum(fused3, axis=1) + pos_c
    h = jax.lax.dot_general(fused, wp_ref[...], (((1,), (1,)), ((), ())),
                            preferred_element_type=jnp.float32) + bp_ref[...]
    h_ref[pl.ds(r0, _TB), :] = h
    out_ptr_ref[pl.ds(r0, _TB), :] = jnp.where(
        jnp.logical_and(mask, ptr == _S - 1), 0, ptr + mask.astype(ptr.dtype))


def kernel(x_t, slots, ptr, W_value, b_value, W_event, b_event, pos_emb,
           slot_weights, W_proj, b_proj):
    B, D = x_t.shape
    S = slots.shape[1]
    H = W_proj.shape[0]
    TB = _TB
    ptr2 = ptr.reshape(B, 1)
    h, new_slots, new_ptr2 = pl.pallas_call(
        _cell_kernel,
        grid=(B // TB,),
        in_specs=[
            pl.BlockSpec((B, D), lambda i: (0, 0)),
            pl.BlockSpec((TB, S, D), lambda i: (i, 0, 0)),
            pl.BlockSpec((B, 1), lambda i: (0, 0)),
            pl.BlockSpec((D, D), lambda i: (0, 0)),
            pl.BlockSpec((1, D), lambda i: (0, 0)),
            pl.BlockSpec((1, D), lambda i: (0, 0)),
            pl.BlockSpec((1, 1), lambda i: (0, 0)),
            pl.BlockSpec((S, D), lambda i: (0, 0)),
            pl.BlockSpec((1, S), lambda i: (0, 0)),
            pl.BlockSpec((H, D), lambda i: (0, 0)),
            pl.BlockSpec((1, H), lambda i: (0, 0)),
        ],
        out_specs=[
            pl.BlockSpec((B, H), lambda i: (0, 0)),
            pl.BlockSpec((TB, S, D), lambda i: (i, 0, 0)),
            pl.BlockSpec((B, 1), lambda i: (0, 0)),
        ],
        out_shape=[
            jax.ShapeDtypeStruct((B, H), jnp.float32),
            jax.ShapeDtypeStruct((B, S, D), jnp.float32),
            jax.ShapeDtypeStruct((B, 1), ptr.dtype),
        ],
        compiler_params=pltpu.CompilerParams(
            dimension_semantics=("parallel",)),
    )(x_t, slots, ptr2, W_value, b_value.reshape(1, D), W_event,
      b_event.reshape(1, 1), pos_emb, slot_weights.reshape(1, S), W_proj,
      b_proj.reshape(1, H))
    return h, new_slots, new_ptr2.reshape(B)
